# Initial kernel scaffold; baseline (speedup 1.0000x reference)
import functools

import jax
import jax.numpy as jnp
from jax import lax
from jax.experimental import pallas as pl
from jax.experimental.pallas import tpu as pltpu
from jax.experimental.pallas import tpu_sc as plsc

_NV = 33
_NV3 = _NV ** 3
_IMG = 512
_BATCH = 4
_NC = 2
_NS = 16
_NW = _NC * _NS
_WORKERS_PER_IMG = _NW // _BATCH
_PX_PER_IMG = _IMG * _IMG
_PX_PER_W = _PX_PER_IMG // _WORKERS_PER_IMG
_CHUNK = 4096
_N_CHUNKS = _PX_PER_W // _CHUNK
_LUT_PAD = 3 * _NV3 + 5
_VERT_PAD = 104


def _locate(vert_v, x, cbase):
    lo = jnp.zeros((16,), jnp.int32)
    alo = jnp.zeros((16,), jnp.float32)
    for s in (16, 8, 4, 2, 1):
        m = lo + s
        v = plsc.load_gather(vert_v, [m + cbase])
        take = x >= v
        lo = jnp.where(take, m, lo)
        alo = jnp.where(take, v, alo)
    ahi = plsc.load_gather(vert_v, [lo + (cbase + 1)])
    f = (x - alo) / (ahi - alo + 1e-10)
    return lo, f


def _ailut_body(imgs_hbm, luts_hbm, verts_hbm, out_hbm, lut_v, vert_v,
                r_v, g_v, b_v):
    wid = lax.axis_index("s") * _NC + lax.axis_index("c")
    img = wid // _WORKERS_PER_IMG
    slot = wid % _WORKERS_PER_IMG

    pltpu.sync_copy(luts_hbm.at[img], lut_v)
    pltpu.sync_copy(verts_hbm.at[img], vert_v)

    base_px = slot * _PX_PER_W

    def group(i, carry):
        p = i * 16
        r = r_v[pl.ds(p, 16)]
        g = g_v[pl.ds(p, 16)]
        b = b_v[pl.ds(p, 16)]
        rid, rf = _locate(vert_v, r, 0)
        gid, gf = _locate(vert_v, g, _NV)
        bid, bf = _locate(vert_v, b, 2 * _NV)
        base3 = bid * (_NV * _NV) + gid * _NV + rid
        wr0 = 1.0 - rf
        wg0 = 1.0 - gf
        wb0 = 1.0 - bf
        w00 = wb0 * wg0
        w01 = wb0 * gf
        w10 = bf * wg0
        w11 = bf * gf
        cw = (
            (0, w00 * wr0), (1, w00 * rf),
            (_NV, w01 * wr0), (_NV + 1, w01 * rf),
            (_NV * _NV, w10 * wr0), (_NV * _NV + 1, w10 * rf),
            (_NV * _NV + _NV, w11 * wr0), (_NV * _NV + _NV + 1, w11 * rf),
        )
        outs = []
        for c in range(3):
            acc = jnp.zeros((16,), jnp.float32)
            for off, w in cw:
                v = plsc.load_gather(lut_v, [base3 + (c * _NV3 + off)])
                acc = acc + w * v
            outs.append(acc)
        r_v[pl.ds(p, 16)] = outs[0]
        g_v[pl.ds(p, 16)] = outs[1]
        b_v[pl.ds(p, 16)] = outs[2]
        return carry

    for chunk in range(_N_CHUNKS):
        off = base_px + chunk * _CHUNK
        pltpu.sync_copy(imgs_hbm.at[img, 0, pl.ds(off, _CHUNK)], r_v)
        pltpu.sync_copy(imgs_hbm.at[img, 1, pl.ds(off, _CHUNK)], g_v)
        pltpu.sync_copy(imgs_hbm.at[img, 2, pl.ds(off, _CHUNK)], b_v)
        lax.fori_loop(0, _CHUNK // 16, group, 0)
        pltpu.sync_copy(r_v, out_hbm.at[img, 0, pl.ds(off, _CHUNK)])
        pltpu.sync_copy(g_v, out_hbm.at[img, 1, pl.ds(off, _CHUNK)])
        pltpu.sync_copy(b_v, out_hbm.at[img, 2, pl.ds(off, _CHUNK)])


def _ailut_sc(imgs_flat, luts_pad, verts_pad):
    mesh = plsc.VectorSubcoreMesh(core_axis_name="c", subcore_axis_name="s")
    run = functools.partial(
        pl.kernel,
        mesh=mesh,
        out_type=jax.ShapeDtypeStruct((_BATCH, 3, _PX_PER_IMG), jnp.float32),
        scratch_types=[
            pltpu.VMEM((_LUT_PAD,), jnp.float32),
            pltpu.VMEM((_VERT_PAD,), jnp.float32),
            pltpu.VMEM((_CHUNK,), jnp.float32),
            pltpu.VMEM((_CHUNK,), jnp.float32),
            pltpu.VMEM((_CHUNK,), jnp.float32),
        ],
    )(_ailut_body)
    return run(imgs_flat, luts_pad, verts_pad)


def _conv(x, w, b, stride):
    y = lax.conv_general_dilated(x, w, (stride, stride),
                                 padding=((1, 1), (1, 1)),
                                 dimension_numbers=('NCHW', 'OIHW', 'NCHW'))
    return y + b[None, :, None, None]


def _inorm(x, g, b, eps=1e-5):
    m = x.mean(axis=(2, 3), keepdims=True)
    v = ((x - m) ** 2).mean(axis=(2, 3), keepdims=True)
    return g[None, :, None, None] * (x - m) / jnp.sqrt(v + eps) + b[None, :, None, None]


def kernel(imgs, conv_w0, conv_b0, conv_w1, conv_b1, conv_w2, conv_b2,
           conv_w3, conv_b3, conv_w4, conv_b4, in_g0, in_b0, in_g1, in_b1,
           in_g2, in_b2, in_g3, in_b3, Wg, bg, Wl, Wa, ba):
    b = imgs.shape[0]
    x = jax.image.resize(imgs, (b, 3, 256, 256), method='bilinear')
    convs = ((conv_w0, conv_b0), (conv_w1, conv_b1), (conv_w2, conv_b2),
             (conv_w3, conv_b3), (conv_w4, conv_b4))
    norms = ((in_g0, in_b0), (in_g1, in_b1), (in_g2, in_b2), (in_g3, in_b3))
    for i in range(5):
        x = _conv(x, convs[i][0], convs[i][1], 2)
        x = jnp.where(x >= 0, x, 0.2 * x)
        if i < 4:
            x = _inorm(x, norms[i][0], norms[i][1])
    bb, cc, hh, ww = x.shape
    x = x.reshape(bb, cc, 2, hh // 2, 2, ww // 2).mean(axis=(3, 5))
    codes = x.reshape(bb, -1)

    weights = codes @ Wg.T + bg
    luts = weights @ Wl.T
    intervals = (codes @ Wa.T + ba).reshape(b, 3, _NV - 1)
    intervals = jax.nn.softmax(intervals, axis=-1)
    vertices = jnp.pad(jnp.cumsum(intervals, axis=-1), ((0, 0), (0, 0), (1, 0)))

    luts_pad = jnp.pad(luts, ((0, 0), (0, _LUT_PAD - 3 * _NV3)))
    verts_pad = jnp.pad(vertices.reshape(b, 3 * _NV),
                        ((0, 0), (0, _VERT_PAD - 3 * _NV)))
    imgs_flat = imgs.reshape(b, 3, _PX_PER_IMG)
    out = _ailut_sc(imgs_flat, luts_pad, verts_pad)
    return out.reshape(b, 3, _IMG, _IMG)


# speedup vs baseline: 1399.8653x; 1399.8653x over previous
import functools

import jax
import jax.numpy as jnp
from jax import lax
from jax.experimental import pallas as pl
from jax.experimental.pallas import tpu as pltpu
from jax.experimental.pallas import tpu_sc as plsc

_NV = 33
_NV3 = _NV ** 3
_IMG = 512
_BATCH = 4
_NC = 2
_NS = 16
_NW = _NC * _NS
_WORKERS_PER_IMG = _NW // _BATCH
_PX_PER_IMG = _IMG * _IMG
_PX_PER_W = _PX_PER_IMG // _WORKERS_PER_IMG
_CHUNK = 4096
_N_CHUNKS = _PX_PER_W // _CHUNK
_LUT_PAD = 3 * _NV3 + 5
_VERT_PAD = 104


def _locate(vert_v, x, cbase):
    lo = jnp.zeros((16,), jnp.int32)
    alo = jnp.zeros((16,), jnp.float32)
    for s in (16, 8, 4, 2, 1):
        m = lo + s
        v = plsc.load_gather(vert_v, [m + cbase])
        take = x >= v
        lo = jnp.where(take, m, lo)
        alo = jnp.where(take, v, alo)
    ahi = plsc.load_gather(vert_v, [lo + (cbase + 1)])
    f = (x - alo) / (ahi - alo + 1e-10)
    return lo, f


def _ailut_body(imgs_hbm, luts_hbm, verts_hbm, out_hbm, lut_v, vert_v,
                r_v, g_v, b_v):
    wid = lax.axis_index("s") * _NC + lax.axis_index("c")
    img = wid // _WORKERS_PER_IMG
    slot = wid % _WORKERS_PER_IMG

    pltpu.sync_copy(luts_hbm.at[pl.ds(img * _LUT_PAD, _LUT_PAD)], lut_v)
    pltpu.sync_copy(verts_hbm.at[pl.ds(img * _VERT_PAD, _VERT_PAD)], vert_v)

    img_base = img * 3 * _PX_PER_IMG
    base_px = slot * _PX_PER_W

    def group(i, carry):
        p = i * 16
        r = r_v[pl.ds(p, 16)]
        g = g_v[pl.ds(p, 16)]
        b = b_v[pl.ds(p, 16)]
        rid, rf = _locate(vert_v, r, 0)
        gid, gf = _locate(vert_v, g, _NV)
        bid, bf = _locate(vert_v, b, 2 * _NV)
        base3 = bid * (_NV * _NV) + gid * _NV + rid
        wr0 = 1.0 - rf
        wg0 = 1.0 - gf
        wb0 = 1.0 - bf
        w00 = wb0 * wg0
        w01 = wb0 * gf
        w10 = bf * wg0
        w11 = bf * gf
        cw = (
            (0, w00 * wr0), (1, w00 * rf),
            (_NV, w01 * wr0), (_NV + 1, w01 * rf),
            (_NV * _NV, w10 * wr0), (_NV * _NV + 1, w10 * rf),
            (_NV * _NV + _NV, w11 * wr0), (_NV * _NV + _NV + 1, w11 * rf),
        )
        outs = []
        for c in range(3):
            acc = jnp.zeros((16,), jnp.float32)
            for off, w in cw:
                v = plsc.load_gather(lut_v, [base3 + (c * _NV3 + off)])
                acc = acc + w * v
            outs.append(acc)
        r_v[pl.ds(p, 16)] = outs[0]
        g_v[pl.ds(p, 16)] = outs[1]
        b_v[pl.ds(p, 16)] = outs[2]
        return carry

    for chunk in range(_N_CHUNKS):
        off = img_base + base_px + chunk * _CHUNK
        pltpu.sync_copy(imgs_hbm.at[pl.ds(off, _CHUNK)], r_v)
        pltpu.sync_copy(imgs_hbm.at[pl.ds(off + _PX_PER_IMG, _CHUNK)], g_v)
        pltpu.sync_copy(imgs_hbm.at[pl.ds(off + 2 * _PX_PER_IMG, _CHUNK)], b_v)
        lax.fori_loop(0, _CHUNK // 16, group, 0)
        pltpu.sync_copy(r_v, out_hbm.at[pl.ds(off, _CHUNK)])
        pltpu.sync_copy(g_v, out_hbm.at[pl.ds(off + _PX_PER_IMG, _CHUNK)])
        pltpu.sync_copy(b_v, out_hbm.at[pl.ds(off + 2 * _PX_PER_IMG, _CHUNK)])


def _ailut_sc(imgs_flat, luts_pad, verts_pad):
    mesh = plsc.VectorSubcoreMesh(core_axis_name="c", subcore_axis_name="s")
    run = functools.partial(
        pl.kernel,
        mesh=mesh,
        compiler_params=pltpu.CompilerParams(needs_layout_passes=False),
        out_type=jax.ShapeDtypeStruct((_BATCH * 3 * _PX_PER_IMG,), jnp.float32),
        scratch_types=[
            pltpu.VMEM((_LUT_PAD,), jnp.float32),
            pltpu.VMEM((_VERT_PAD,), jnp.float32),
            pltpu.VMEM((_CHUNK,), jnp.float32),
            pltpu.VMEM((_CHUNK,), jnp.float32),
            pltpu.VMEM((_CHUNK,), jnp.float32),
        ],
    )(_ailut_body)
    return run(imgs_flat, luts_pad, verts_pad)


def _conv(x, w, b, stride):
    y = lax.conv_general_dilated(x, w, (stride, stride),
                                 padding=((1, 1), (1, 1)),
                                 dimension_numbers=('NCHW', 'OIHW', 'NCHW'))
    return y + b[None, :, None, None]


def _inorm(x, g, b, eps=1e-5):
    m = x.mean(axis=(2, 3), keepdims=True)
    v = ((x - m) ** 2).mean(axis=(2, 3), keepdims=True)
    return g[None, :, None, None] * (x - m) / jnp.sqrt(v + eps) + b[None, :, None, None]


def kernel(imgs, conv_w0, conv_b0, conv_w1, conv_b1, conv_w2, conv_b2,
           conv_w3, conv_b3, conv_w4, conv_b4, in_g0, in_b0, in_g1, in_b1,
           in_g2, in_b2, in_g3, in_b3, Wg, bg, Wl, Wa, ba):
    b = imgs.shape[0]
    x = jax.image.resize(imgs, (b, 3, 256, 256), method='bilinear')
    convs = ((conv_w0, conv_b0), (conv_w1, conv_b1), (conv_w2, conv_b2),
             (conv_w3, conv_b3), (conv_w4, conv_b4))
    norms = ((in_g0, in_b0), (in_g1, in_b1), (in_g2, in_b2), (in_g3, in_b3))
    for i in range(5):
        x = _conv(x, convs[i][0], convs[i][1], 2)
        x = jnp.where(x >= 0, x, 0.2 * x)
        if i < 4:
            x = _inorm(x, norms[i][0], norms[i][1])
    bb, cc, hh, ww = x.shape
    x = x.reshape(bb, cc, 2, hh // 2, 2, ww // 2).mean(axis=(3, 5))
    codes = x.reshape(bb, -1)

    weights = codes @ Wg.T + bg
    luts = weights @ Wl.T
    intervals = (codes @ Wa.T + ba).reshape(b, 3, _NV - 1)
    intervals = jax.nn.softmax(intervals, axis=-1)
    vertices = jnp.pad(jnp.cumsum(intervals, axis=-1), ((0, 0), (0, 0), (1, 0)))

    luts_pad = jnp.pad(luts, ((0, 0), (0, _LUT_PAD - 3 * _NV3))).reshape(-1)
    verts_pad = jnp.pad(vertices.reshape(b, 3 * _NV),
                        ((0, 0), (0, _VERT_PAD - 3 * _NV))).reshape(-1)
    imgs_flat = imgs.reshape(-1)
    out = _ailut_sc(imgs_flat, luts_pad, verts_pad)
    return out.reshape(b, 3, _IMG, _IMG)
